# baseline (speedup 1.0000x reference)
import functools

import jax
import jax.numpy as jnp
from jax.experimental import pallas as pl
from jax.experimental.pallas import tpu as pltpu

EPS = 1e-09


def _logmse_block(y_pred_ref, y_true_ref, out_ref):
    yp = y_pred_ref[...]
    yt = y_true_ref[...]
    lp = jnp.log2(jnp.maximum(yp, EPS))
    lt = jnp.log2(jnp.maximum(yt, EPS))
    d = lt - lp
    partial = jnp.sum(d * d).reshape(1, 1)
    out_ref[...] = jnp.broadcast_to(partial, out_ref.shape)


def kernel(y_pred, y_true):
    b, s, _ = y_pred.shape
    lanes = 128
    reps = s // lanes
    n = b * reps
    yp = y_pred.reshape(n, lanes)
    yt = jnp.broadcast_to(y_true.reshape(b, 1, 1), (b, reps, 1)).reshape(n, 1)
    rows = 2048
    nblocks = n // rows
    inv_n = 1.0 / float(b * s)
    out = pl.pallas_call(
        _logmse_block,
        grid=(nblocks,),
        in_specs=[
            pl.BlockSpec((rows, lanes), lambda i: (i, 0)),
            pl.BlockSpec((rows, 1), lambda i: (i, 0)),
        ],
        out_specs=pl.BlockSpec((1, 1, lanes), lambda i: (i, 0, 0)),
        out_shape=jax.ShapeDtypeStruct((nblocks, 1, lanes), jnp.float32),
        compiler_params=pltpu.CompilerParams(
            dimension_semantics=("parallel",),
        ),
    )(yp, yt)
    loss = jnp.sum(out[:, 0, 0]) * inv_n
    return (loss, loss, jnp.array(0.0, dtype=jnp.float32))
